# baseline (speedup 1.0000x reference)
import math

import jax
import jax.numpy as jnp
from jax import lax
from jax.experimental import pallas as pl
from jax.experimental.pallas import tpu as pltpu
from jax.experimental.pallas import tpu_sc as plsc

VOCAB = 1000000
HIDDEN = 64
BATCH = 4096
HIST = 200

NC = 2
NS = 16
NW = NC * NS
LANES = 16

B_TOTAL = BATCH * HIST
B_PER_W = B_TOTAL // NW
CHUNK = 128
NCHUNK = B_PER_W // CHUNK
NBUF = 4

EMB_SCALE = math.sqrt(HIDDEN)


def _sc_body(idx_hbm, table_hbm, out_hbm, idx_all, ins, outs, gsems, osems):
    wid = lax.axis_index("s") * NC + lax.axis_index("c")

    pltpu.sync_copy(idx_hbm.at[wid], idx_all)

    base = wid * B_PER_W

    def start_gather(g, b):
        pltpu.make_async_copy(
            table_hbm.at[idx_all.at[g]], ins[b], gsems[b]
        ).start()

    def wait_gather(b):
        pltpu.make_async_copy(
            table_hbm.at[idx_all.at[0]], ins[b], gsems[b]
        ).wait()

    def start_write(g, b):
        pltpu.make_async_copy(
            outs[b], out_hbm.at[pl.ds(base + g * CHUNK, CHUNK)], osems[b]
        ).start()

    def wait_write(b):
        pltpu.make_async_copy(
            outs[b], out_hbm.at[pl.ds(base, CHUNK)], osems[b]
        ).wait()

    def scale(b):
        src, dst = ins[b], outs[b]

        def row_step(r, _):
            for c in range(HIDDEN // LANES):
                sl = pl.ds(c * LANES, LANES)
                dst[r, sl] = src[r, sl] * EMB_SCALE
            return 0

        lax.fori_loop(0, CHUNK, row_step, 0, unroll=8)

    for g in range(NBUF):
        start_gather(g, g)
    for g in range(NBUF):
        b = g
        wait_gather(b)
        scale(b)
        start_write(g, b)
        start_gather(g + NBUF, b)

    def steady(i, _):
        g0 = NBUF + NBUF * i
        for b in range(NBUF):
            g = g0 + b
            wait_gather(b)
            wait_write(b)
            scale(b)
            start_write(g, b)
            start_gather(g + NBUF, b)
        return 0

    lax.fori_loop(0, (NCHUNK - 2 * NBUF) // NBUF, steady, 0)

    for g in range(NCHUNK - NBUF, NCHUNK):
        b = g % NBUF
        wait_gather(b)
        wait_write(b)
        scale(b)
        start_write(g, b)
    for b in range(NBUF):
        wait_write(b)


@jax.jit
def _emb_lookup(idx_grouped, table):
    mesh = plsc.VectorSubcoreMesh(core_axis_name="c", subcore_axis_name="s")
    run = pl.kernel(
        _sc_body,
        out_type=jax.ShapeDtypeStruct((B_TOTAL, HIDDEN), jnp.float32),
        mesh=mesh,
        scratch_types=[
            pltpu.VMEM((NCHUNK, CHUNK), jnp.int32),
            [pltpu.VMEM((CHUNK, HIDDEN), jnp.float32) for _ in range(NBUF)],
            [pltpu.VMEM((CHUNK, HIDDEN), jnp.float32) for _ in range(NBUF)],
            [pltpu.SemaphoreType.DMA for _ in range(NBUF)],
            [pltpu.SemaphoreType.DMA for _ in range(NBUF)],
        ],
        compiler_params=pltpu.CompilerParams(use_tc_tiling_on_sc=False),
    )
    return run(idx_grouped, table)


def kernel(input_ids, table):
    idx_grouped = input_ids.reshape(NW, NCHUNK, CHUNK).astype(jnp.int32)
    out = _emb_lookup(idx_grouped, table)
    return out.reshape(BATCH, HIST, HIDDEN)


# speedup vs baseline: 1.3101x; 1.3101x over previous
import math

import jax
import jax.numpy as jnp
from jax import lax
from jax.experimental import pallas as pl
from jax.experimental.pallas import tpu as pltpu
from jax.experimental.pallas import tpu_sc as plsc

VOCAB = 1000000
HIDDEN = 64
PADDED = 128
BATCH = 4096
HIST = 200

NC = 2
NS = 16
NW = NC * NS
LANES = 16

B_TOTAL = BATCH * HIST
B_PER_W = B_TOTAL // NW
CHUNK = 128
NCHUNK = B_PER_W // CHUNK
NBUF = 5

EMB_SCALE = math.sqrt(HIDDEN)


def _sc_body(idx_hbm, table_hbm, out_hbm, idx_all, ins, outs, gsems, osems):
    wid = lax.axis_index("s") * NC + lax.axis_index("c")

    pltpu.sync_copy(idx_hbm.at[wid], idx_all)

    base = wid * B_PER_W

    def start_gather(g, b):
        pltpu.make_async_copy(
            table_hbm.at[idx_all.at[g]], ins[b], gsems[b]
        ).start()

    def wait_gather(b):
        pltpu.make_async_copy(
            table_hbm.at[idx_all.at[0]], ins[b], gsems[b]
        ).wait()

    def start_write(g, b):
        pltpu.make_async_copy(
            outs[b],
            out_hbm.at[pl.ds(base + g * CHUNK, CHUNK), pl.ds(0, HIDDEN)],
            osems[b],
        ).start()

    def wait_write(b):
        pltpu.make_async_copy(
            outs[b],
            out_hbm.at[pl.ds(base, CHUNK), pl.ds(0, HIDDEN)],
            osems[b],
        ).wait()

    def scale(b):
        src, dst = ins[b], outs[b]

        def row_step(r, _):
            for c in range(HIDDEN // LANES):
                sl = pl.ds(c * LANES, LANES)
                dst[r, sl] = src[r, sl] * EMB_SCALE
            return 0

        lax.fori_loop(0, CHUNK, row_step, 0, unroll=8)

    for g in range(NBUF):
        start_gather(g, g)
    for g in range(NBUF):
        b = g
        wait_gather(b)
        scale(b)
        start_write(g, b)
        start_gather(g + NBUF, b)

    def steady(i, _):
        g0 = NBUF + NBUF * i
        for b in range(NBUF):
            g = g0 + b
            wait_gather(b)
            wait_write(b)
            scale(b)
            start_write(g, b)
            start_gather(g + NBUF, b)
        return 0

    lax.fori_loop(0, (NCHUNK - 2 * NBUF) // NBUF, steady, 0)

    for g in range(NCHUNK - NBUF, NCHUNK):
        b = g % NBUF
        wait_gather(b)
        wait_write(b)
        scale(b)
        start_write(g, b)
    for b in range(NBUF):
        wait_write(b)


@jax.jit
def _emb_lookup(idx_grouped, table_rows):
    mesh = plsc.VectorSubcoreMesh(core_axis_name="c", subcore_axis_name="s")
    run = pl.kernel(
        _sc_body,
        out_type=jax.ShapeDtypeStruct((B_TOTAL, PADDED), jnp.float32),
        mesh=mesh,
        scratch_types=[
            pltpu.VMEM((NCHUNK, CHUNK), jnp.int32),
            [pltpu.VMEM((CHUNK, HIDDEN), jnp.float32) for _ in range(NBUF)],
            [pltpu.VMEM((CHUNK, HIDDEN), jnp.float32) for _ in range(NBUF)],
            [pltpu.SemaphoreType.DMA for _ in range(NBUF)],
            [pltpu.SemaphoreType.DMA for _ in range(NBUF)],
        ],
        compiler_params=pltpu.CompilerParams(use_tc_tiling_on_sc=False),
    )
    return run(idx_grouped, table_rows)


def kernel(input_ids, table):
    table_rows = jnp.pad(table, ((0, 0), (0, PADDED - HIDDEN))).reshape(
        2 * VOCAB, HIDDEN
    )
    idx_grouped = (input_ids.astype(jnp.int32) * 2).reshape(NW, NCHUNK, CHUNK)
    out = _emb_lookup(idx_grouped, table_rows)
    return out.reshape(BATCH, HIST, PADDED)[..., :HIDDEN]


# speedup vs baseline: 1.8102x; 1.3817x over previous
import math

import jax
import jax.numpy as jnp
from jax import lax
from jax.experimental import pallas as pl
from jax.experimental.pallas import tpu as pltpu
from jax.experimental.pallas import tpu_sc as plsc

VOCAB = 1000000
HIDDEN = 64
PADDED = 128
BATCH = 4096
HIST = 200

NC = 2
NS = 16
NW = NC * NS
LANES = 16

B_TOTAL = BATCH * HIST
B_PER_W = B_TOTAL // NW
CHUNK = 128
NCHUNK = B_PER_W // CHUNK
NBUF = 8

EMB_SCALE = math.sqrt(HIDDEN)


def _sc_body(idx_hbm, table_hbm, out_hbm, idx_all, bufs, gsems, osems):
    wid = lax.axis_index("s") * NC + lax.axis_index("c")

    pltpu.sync_copy(idx_hbm.at[wid], idx_all)

    base = wid * B_PER_W

    def start_gather(g, b):
        pltpu.make_async_copy(
            table_hbm.at[idx_all.at[g]], bufs[b], gsems[b]
        ).start()

    def wait_gather(b):
        pltpu.make_async_copy(
            table_hbm.at[idx_all.at[0]], bufs[b], gsems[b]
        ).wait()

    def start_write(g, b):
        pltpu.make_async_copy(
            bufs[b],
            out_hbm.at[pl.ds(base + g * CHUNK, CHUNK), pl.ds(0, HIDDEN)],
            osems[b],
        ).start()

    def wait_write(b):
        pltpu.make_async_copy(
            bufs[b],
            out_hbm.at[pl.ds(base, CHUNK), pl.ds(0, HIDDEN)],
            osems[b],
        ).wait()

    for g in range(NBUF):
        start_gather(g, g)
    for g in range(NBUF):
        wait_gather(g)
        start_write(g, g)

    def steady(i, _):
        g0 = NBUF + NBUF * i
        for b in range(NBUF):
            g = g0 + b
            wait_write(b)
            start_gather(g, b)
        for b in range(NBUF):
            g = g0 + b
            wait_gather(b)
            start_write(g, b)
        return 0

    lax.fori_loop(0, NCHUNK // NBUF - 1, steady, 0)

    for b in range(NBUF):
        wait_write(b)


@jax.jit
def _emb_lookup(idx_grouped, table_rows):
    mesh = plsc.VectorSubcoreMesh(core_axis_name="c", subcore_axis_name="s")
    run = pl.kernel(
        _sc_body,
        out_type=jax.ShapeDtypeStruct((B_TOTAL, PADDED), jnp.float32),
        mesh=mesh,
        scratch_types=[
            pltpu.VMEM((NCHUNK, CHUNK), jnp.int32),
            [pltpu.VMEM((CHUNK, HIDDEN), jnp.float32) for _ in range(NBUF)],
            [pltpu.SemaphoreType.DMA for _ in range(NBUF)],
            [pltpu.SemaphoreType.DMA for _ in range(NBUF)],
        ],
        compiler_params=pltpu.CompilerParams(use_tc_tiling_on_sc=False),
    )
    return run(idx_grouped, table_rows)


def kernel(input_ids, table):
    table_rows = (
        jnp.pad(table, ((0, 0), (0, PADDED - HIDDEN))) * EMB_SCALE
    ).reshape(2 * VOCAB, HIDDEN)
    idx_grouped = (input_ids.astype(jnp.int32) * 2).reshape(NW, NCHUNK, CHUNK)
    out = _emb_lookup(idx_grouped, table_rows)
    return out.reshape(BATCH, HIST, PADDED)[..., :HIDDEN]
